# baseline (speedup 1.0000x reference)
import functools

import jax
import jax.numpy as jnp
from jax import lax
from jax.experimental import pallas as pl
from jax.experimental.pallas import tpu as pltpu
from jax.experimental.pallas import tpu_sc as plsc

_LBW = 0.01
_KT = 256
_SC_CHUNK = 256
_NC, _NS = 2, 16


def _sc_colsums(x):
    B, S, D = x.shape
    cols = 128
    ngrp = cols // 16
    ngcol = D // cols
    half = S // 2
    nch = half // _SC_CHUNK
    total = B * nch
    mesh = plsc.VectorSubcoreMesh(core_axis_name="c", subcore_axis_name="s")

    @functools.partial(
        pl.kernel, mesh=mesh,
        out_type=jax.ShapeDtypeStruct((2, B, D), jnp.float32),
        scratch_types=[
            pltpu.VMEM((_SC_CHUNK, cols), jnp.float32),
            pltpu.VMEM((_SC_CHUNK, cols), jnp.float32),
            pltpu.VMEM((B, cols), jnp.float32),
            pltpu.SemaphoreType.DMA,
            pltpu.SemaphoreType.DMA,
        ],
    )
    def colsums(x_hbm, out_hbm, buf0, buf1, accv, sem0, sem1):
        wid = lax.axis_index("s") * _NC + lax.axis_index("c")
        g = wid % ngcol
        h = wid // ngcol
        d0 = g * cols
        s_base = h * half
        bufs = (buf0, buf1)
        sems = (sem0, sem1)

        def srcslice(t):
            b, c = divmod(t, nch)
            return x_hbm.at[b, pl.ds(s_base + c * _SC_CHUNK, _SC_CHUNK),
                            pl.ds(d0, cols)]

        copies = [pltpu.async_copy(srcslice(0), bufs[0], sems[0]), None]
        acc = None
        for t in range(total):
            b, c = divmod(t, nch)
            if t + 1 < total:
                copies[(t + 1) % 2] = pltpu.async_copy(
                    srcslice(t + 1), bufs[(t + 1) % 2], sems[(t + 1) % 2])
            copies[t % 2].wait()
            buf = bufs[t % 2]
            if c == 0:
                acc = tuple(jnp.zeros((16,), jnp.float32)
                            for _ in range(ngrp))

            def body(i, carry, buf=buf):
                accs = list(carry)
                for u in range(4):
                    for gg in range(ngrp):
                        accs[gg] = accs[gg] + buf[i * 4 + u,
                                                  pl.ds(gg * 16, 16)]
                return tuple(accs)

            acc = lax.fori_loop(0, _SC_CHUNK // 4, body, acc)
            if c == nch - 1:
                for gg in range(ngrp):
                    accv[b, pl.ds(gg * 16, 16)] = acc[gg]
        pltpu.sync_copy(accv,
                        out_hbm.at[h, pl.ds(0, B), pl.ds(d0, cols)])

    return colsums(x)


def _tc_gate_kernel(sums_ref, w1_ref, b1_ref, w2_ref, b2_ref,
                    wts_ref, idx_ref, loss_ref, acc_ref, *, nk, s, e, topk):
    k = pl.program_id(0)

    @pl.when(k == 0)
    def _init():
        acc_ref[...] = jnp.zeros_like(acc_ref)

    r = (sums_ref[0] + sums_ref[1]) * (1.0 / s)
    acc_ref[...] += jnp.dot(r, w1_ref[...],
                            preferred_element_type=jnp.float32)

    @pl.when(k == nk - 1)
    def _epilogue():
        h = acc_ref[...] + b1_ref[...]
        h = h * jax.nn.sigmoid(h)
        logits = jnp.dot(h, w2_ref[...],
                         preferred_element_type=jnp.float32) + b2_ref[...]
        b = logits.shape[0]
        iota_e = jax.lax.broadcasted_iota(jnp.int32, (b, e), 1)
        m1 = jnp.max(logits, axis=-1)
        i1 = jnp.argmax(logits, axis=-1).astype(jnp.int32)
        masked = jnp.where(iota_e == i1[:, None], -jnp.inf, logits)
        m2 = jnp.max(masked, axis=-1)
        i2 = jnp.argmax(masked, axis=-1).astype(jnp.int32)
        e2 = jnp.exp(m2 - m1)
        denom = 1.0 + e2
        w_first = 1.0 / denom
        w_second = e2 / denom
        iota2 = jax.lax.broadcasted_iota(jnp.int32, (b, topk), 1)
        wts_ref[...] = jnp.where(iota2 == 0, w_first[:, None], w_second[:, None])
        idx_ref[...] = jnp.where(iota2 == 0, i1[:, None], i2[:, None])
        probs = jax.nn.softmax(logits, axis=-1)
        mean_prob = jnp.mean(probs, axis=0)
        onehot = ((iota_e == i1[:, None]).astype(jnp.float32)
                  + (iota_e == i2[:, None]).astype(jnp.float32))
        usage = jnp.sum(onehot, axis=0)
        mean_usage = usage / (b * topk)
        loss = _LBW * e * jnp.sum(mean_prob * mean_usage)
        loss_ref[...] = loss[None, None]


def _tc_gate(sums, W1, b1, W2, b2, s):
    _, B, D = sums.shape
    E = W2.shape[1]
    TOPK = 2
    nk = D // _KT
    kfn = functools.partial(_tc_gate_kernel, nk=nk, s=s, e=E, topk=TOPK)
    wts, idx, loss = pl.pallas_call(
        kfn,
        grid=(nk,),
        in_specs=[
            pl.BlockSpec((2, B, _KT), lambda k: (0, 0, k)),
            pl.BlockSpec((_KT, D), lambda k: (k, 0)),
            pl.BlockSpec((1, D), lambda k: (0, 0)),
            pl.BlockSpec((D, E), lambda k: (0, 0)),
            pl.BlockSpec((1, E), lambda k: (0, 0)),
        ],
        out_specs=[
            pl.BlockSpec((B, TOPK), lambda k: (0, 0)),
            pl.BlockSpec((B, TOPK), lambda k: (0, 0)),
            pl.BlockSpec((1, 1), lambda k: (0, 0)),
        ],
        out_shape=[
            jax.ShapeDtypeStruct((B, TOPK), jnp.float32),
            jax.ShapeDtypeStruct((B, TOPK), jnp.int32),
            jax.ShapeDtypeStruct((1, 1), jnp.float32),
        ],
        scratch_shapes=[pltpu.VMEM((B, D), jnp.float32)],
        compiler_params=pltpu.CompilerParams(
            dimension_semantics=("arbitrary",),
        ),
    )(sums, W1, b1.reshape(1, D), W2, b2.reshape(1, E))
    return wts, idx, loss.reshape(())


def kernel(x, W1, b1, W2, b2):
    B, S, D = x.shape
    sums = _sc_colsums(x)
    return _tc_gate(sums, W1, b1, W2, b2, S)


# speedup vs baseline: 1.2980x; 1.2980x over previous
import functools

import jax
import jax.numpy as jnp
from jax import lax
from jax.experimental import pallas as pl
from jax.experimental.pallas import tpu as pltpu
from jax.experimental.pallas import tpu_sc as plsc

_LBW = 0.01
_KT = 256
_D_SC = 512
_SC_COLS = 128
_SC_CHUNK = 256
_NC, _NS = 2, 16


def _sc_colsums_tail(x):
    B, S, D = x.shape
    d_base = D - _D_SC
    ngcol = _D_SC // _SC_COLS
    ngrp = _SC_COLS // 16
    nsplit = (_NC * _NS) // ngcol
    rows = S // nsplit
    nch = rows // _SC_CHUNK
    total = B * nch
    mesh = plsc.VectorSubcoreMesh(core_axis_name="c", subcore_axis_name="s")

    @functools.partial(
        pl.kernel, mesh=mesh,
        out_type=jax.ShapeDtypeStruct((nsplit, B, _D_SC), jnp.float32),
        scratch_types=[
            pltpu.VMEM((_SC_CHUNK, _SC_COLS), jnp.float32),
            pltpu.VMEM((_SC_CHUNK, _SC_COLS), jnp.float32),
            pltpu.VMEM((B, _SC_COLS), jnp.float32),
            pltpu.SemaphoreType.DMA,
            pltpu.SemaphoreType.DMA,
        ],
    )
    def colsums(x_hbm, out_hbm, buf0, buf1, accv, sem0, sem1):
        wid = lax.axis_index("s") * _NC + lax.axis_index("c")
        g = wid % ngcol
        p = wid // ngcol
        d0 = d_base + g * _SC_COLS
        s_base = p * rows
        bufs = (buf0, buf1)
        sems = (sem0, sem1)

        def srcslice(t):
            b, c = divmod(t, nch)
            return x_hbm.at[b, pl.ds(s_base + c * _SC_CHUNK, _SC_CHUNK),
                            pl.ds(d0, _SC_COLS)]

        copies = [pltpu.async_copy(srcslice(0), bufs[0], sems[0]), None]
        acc = None
        for t in range(total):
            b, c = divmod(t, nch)
            if t + 1 < total:
                copies[(t + 1) % 2] = pltpu.async_copy(
                    srcslice(t + 1), bufs[(t + 1) % 2], sems[(t + 1) % 2])
            copies[t % 2].wait()
            buf = bufs[t % 2]
            if c == 0:
                acc = tuple(jnp.zeros((16,), jnp.float32)
                            for _ in range(ngrp))

            def body(i, carry, buf=buf):
                accs = list(carry)
                for u in range(4):
                    for gg in range(ngrp):
                        accs[gg] = accs[gg] + buf[i * 4 + u,
                                                  pl.ds(gg * 16, 16)]
                return tuple(accs)

            acc = lax.fori_loop(0, _SC_CHUNK // 4, body, acc)
            if c == nch - 1:
                for gg in range(ngrp):
                    accv[b, pl.ds(gg * 16, 16)] = acc[gg]
        pltpu.sync_copy(accv,
                        out_hbm.at[p, pl.ds(0, B), pl.ds(g * _SC_COLS,
                                                         _SC_COLS)])

    return colsums(x)


def _tc_head_kernel(x_ref, w1_ref, acc_ref, *, s):
    k = pl.program_id(0)
    r = jnp.sum(x_ref[...], axis=1) * (1.0 / s)
    part = jnp.dot(r, w1_ref[...], preferred_element_type=jnp.float32)

    @pl.when(k == 0)
    def _first():
        acc_ref[...] = part

    @pl.when(k != 0)
    def _rest():
        acc_ref[...] += part


def _tc_head(x, W1, s):
    B, S, D = x.shape
    nk = (D - _D_SC) // _KT
    kfn = functools.partial(_tc_head_kernel, s=s)
    return pl.pallas_call(
        kfn,
        grid=(nk,),
        in_specs=[
            pl.BlockSpec((B, S, _KT), lambda k: (0, 0, k)),
            pl.BlockSpec((_KT, D), lambda k: (k, 0)),
        ],
        out_specs=pl.BlockSpec((B, D), lambda k: (0, 0)),
        out_shape=jax.ShapeDtypeStruct((B, D), jnp.float32),
        compiler_params=pltpu.CompilerParams(
            dimension_semantics=("arbitrary",),
        ),
    )(x, W1)


def _tc_merge_kernel(acc_head_ref, sums_ref, w1_ref, b1_ref, w2_ref, b2_ref,
                     wts_ref, idx_ref, loss_ref, acc_ref,
                     *, nk, s, e, topk, nsplit):
    j = pl.program_id(0)

    @pl.when(j == 0)
    def _init():
        acc_ref[...] = acc_head_ref[...]

    r = sums_ref[0]
    for p in range(1, nsplit):
        r = r + sums_ref[p]
    r = r * (1.0 / s)
    acc_ref[...] += jnp.dot(r, w1_ref[...],
                            preferred_element_type=jnp.float32)

    @pl.when(j == nk - 1)
    def _epilogue():
        h = acc_ref[...] + b1_ref[...]
        h = h * jax.nn.sigmoid(h)
        logits = jnp.dot(h, w2_ref[...],
                         preferred_element_type=jnp.float32) + b2_ref[...]
        b = logits.shape[0]
        iota_e = jax.lax.broadcasted_iota(jnp.int32, (b, e), 1)
        m1 = jnp.max(logits, axis=-1)
        i1 = jnp.argmax(logits, axis=-1).astype(jnp.int32)
        masked = jnp.where(iota_e == i1[:, None], -jnp.inf, logits)
        m2 = jnp.max(masked, axis=-1)
        i2 = jnp.argmax(masked, axis=-1).astype(jnp.int32)
        e2 = jnp.exp(m2 - m1)
        denom = 1.0 + e2
        w_first = 1.0 / denom
        w_second = e2 / denom
        iota2 = jax.lax.broadcasted_iota(jnp.int32, (b, topk), 1)
        wts_ref[...] = jnp.where(iota2 == 0, w_first[:, None], w_second[:, None])
        idx_ref[...] = jnp.where(iota2 == 0, i1[:, None], i2[:, None])
        probs = jax.nn.softmax(logits, axis=-1)
        mean_prob = jnp.mean(probs, axis=0)
        onehot = ((iota_e == i1[:, None]).astype(jnp.float32)
                  + (iota_e == i2[:, None]).astype(jnp.float32))
        usage = jnp.sum(onehot, axis=0)
        mean_usage = usage / (b * topk)
        loss = _LBW * e * jnp.sum(mean_prob * mean_usage)
        loss_ref[...] = loss[None, None]


def _tc_merge(acc_head, sums, W1, b1, W2, b2, s):
    nsplit, B, _ = sums.shape
    D = W1.shape[0]
    E = W2.shape[1]
    TOPK = 2
    nk = _D_SC // _KT
    nk_head = (D - _D_SC) // _KT
    kfn = functools.partial(_tc_merge_kernel, nk=nk, s=s, e=E, topk=TOPK,
                            nsplit=nsplit)
    wts, idx, loss = pl.pallas_call(
        kfn,
        grid=(nk,),
        in_specs=[
            pl.BlockSpec((B, D), lambda j: (0, 0)),
            pl.BlockSpec((nsplit, B, _KT), lambda j: (0, 0, j)),
            pl.BlockSpec((_KT, D), lambda j: (j + nk_head, 0)),
            pl.BlockSpec((1, D), lambda j: (0, 0)),
            pl.BlockSpec((D, E), lambda j: (0, 0)),
            pl.BlockSpec((1, E), lambda j: (0, 0)),
        ],
        out_specs=[
            pl.BlockSpec((B, TOPK), lambda j: (0, 0)),
            pl.BlockSpec((B, TOPK), lambda j: (0, 0)),
            pl.BlockSpec((1, 1), lambda j: (0, 0)),
        ],
        out_shape=[
            jax.ShapeDtypeStruct((B, TOPK), jnp.float32),
            jax.ShapeDtypeStruct((B, TOPK), jnp.int32),
            jax.ShapeDtypeStruct((1, 1), jnp.float32),
        ],
        scratch_shapes=[pltpu.VMEM((B, D), jnp.float32)],
        compiler_params=pltpu.CompilerParams(
            dimension_semantics=("arbitrary",),
        ),
    )(acc_head, sums, W1, b1.reshape(1, D), W2, b2.reshape(1, E))
    return wts, idx, loss.reshape(())


def kernel(x, W1, b1, W2, b2):
    B, S, D = x.shape
    sums_tail = _sc_colsums_tail(x)
    acc_head = _tc_head(x, W1, S)
    return _tc_merge(acc_head, sums_tail, W1, b1, W2, b2, S)


# speedup vs baseline: 1.3135x; 1.0119x over previous
import functools

import jax
import jax.numpy as jnp
from jax import lax
from jax.experimental import pallas as pl
from jax.experimental.pallas import tpu as pltpu
from jax.experimental.pallas import tpu_sc as plsc

_LBW = 0.01
_KT = 256
_D_SC = 512
_SC_COLS = 128
_SC_CHUNK = 256
_NC, _NS = 2, 16


def _sc_colsums_tail(x):
    B, S, D = x.shape
    d_base = D - _D_SC
    ngcol = _D_SC // _SC_COLS
    ngrp = _SC_COLS // 16
    nsplit = (_NC * _NS) // ngcol
    rows = S // nsplit
    nch = rows // _SC_CHUNK
    total = B * nch
    mesh = plsc.VectorSubcoreMesh(core_axis_name="c", subcore_axis_name="s")

    @functools.partial(
        pl.kernel, mesh=mesh,
        out_type=jax.ShapeDtypeStruct((nsplit, B, _D_SC), jnp.float32),
        scratch_types=[
            pltpu.VMEM((_SC_CHUNK, _SC_COLS), jnp.float32),
            pltpu.VMEM((_SC_CHUNK, _SC_COLS), jnp.float32),
            pltpu.VMEM((B, _SC_COLS), jnp.float32),
            pltpu.SemaphoreType.DMA,
            pltpu.SemaphoreType.DMA,
        ],
    )
    def colsums(x_hbm, out_hbm, buf0, buf1, accv, sem0, sem1):
        wid = lax.axis_index("s") * _NC + lax.axis_index("c")
        g = wid % ngcol
        p = wid // ngcol
        d0 = d_base + g * _SC_COLS
        s_base = p * rows
        bufs = (buf0, buf1)
        sems = (sem0, sem1)

        def srcslice(t):
            b, c = divmod(t, nch)
            return x_hbm.at[b, pl.ds(s_base + c * _SC_CHUNK, _SC_CHUNK),
                            pl.ds(d0, _SC_COLS)]

        copies = [pltpu.async_copy(srcslice(0), bufs[0], sems[0]), None]
        acc = None
        for t in range(total):
            b, c = divmod(t, nch)
            if t + 1 < total:
                copies[(t + 1) % 2] = pltpu.async_copy(
                    srcslice(t + 1), bufs[(t + 1) % 2], sems[(t + 1) % 2])
            copies[t % 2].wait()
            buf = bufs[t % 2]
            if c == 0:
                acc = tuple(jnp.zeros((16,), jnp.float32)
                            for _ in range(ngrp))

            def body(i, carry, buf=buf):
                accs = list(carry)
                for gg in range(ngrp):
                    accs[gg] = accs[gg] + buf[i, pl.ds(gg * 16, 16)]
                return tuple(accs)

            acc = lax.fori_loop(0, _SC_CHUNK, body, acc)
            if c == nch - 1:
                for gg in range(ngrp):
                    accv[b, pl.ds(gg * 16, 16)] = acc[gg]
        pltpu.sync_copy(accv,
                        out_hbm.at[p, pl.ds(0, B), pl.ds(g * _SC_COLS,
                                                         _SC_COLS)])

    return colsums(x)


def _tc_head_kernel(x_ref, w1_ref, acc_ref, *, s):
    k = pl.program_id(0)
    r = jnp.sum(x_ref[...], axis=1) * (1.0 / s)
    part = jnp.dot(r, w1_ref[...], preferred_element_type=jnp.float32)

    @pl.when(k == 0)
    def _first():
        acc_ref[...] = part

    @pl.when(k != 0)
    def _rest():
        acc_ref[...] += part


def _tc_head(x, W1, s):
    B, S, D = x.shape
    nk = (D - _D_SC) // _KT
    kfn = functools.partial(_tc_head_kernel, s=s)
    return pl.pallas_call(
        kfn,
        grid=(nk,),
        in_specs=[
            pl.BlockSpec((B, S, _KT), lambda k: (0, 0, k)),
            pl.BlockSpec((_KT, D), lambda k: (k, 0)),
        ],
        out_specs=pl.BlockSpec((B, D), lambda k: (0, 0)),
        out_shape=jax.ShapeDtypeStruct((B, D), jnp.float32),
        compiler_params=pltpu.CompilerParams(
            dimension_semantics=("arbitrary",),
        ),
    )(x, W1)


def _tc_merge_kernel(acc_head_ref, sums_ref, w1_ref, b1_ref, w2_ref, b2_ref,
                     wts_ref, idx_ref, loss_ref, acc_ref,
                     *, nk, s, e, topk, nsplit):
    j = pl.program_id(0)

    @pl.when(j == 0)
    def _init():
        acc_ref[...] = acc_head_ref[...]

    r = sums_ref[0]
    for p in range(1, nsplit):
        r = r + sums_ref[p]
    r = r * (1.0 / s)
    acc_ref[...] += jnp.dot(r, w1_ref[...],
                            preferred_element_type=jnp.float32)

    @pl.when(j == nk - 1)
    def _epilogue():
        h = acc_ref[...] + b1_ref[...]
        h = h * jax.nn.sigmoid(h)
        logits = jnp.dot(h, w2_ref[...],
                         preferred_element_type=jnp.float32) + b2_ref[...]
        b = logits.shape[0]
        iota_e = jax.lax.broadcasted_iota(jnp.int32, (b, e), 1)
        m1 = jnp.max(logits, axis=-1)
        i1 = jnp.argmax(logits, axis=-1).astype(jnp.int32)
        masked = jnp.where(iota_e == i1[:, None], -jnp.inf, logits)
        m2 = jnp.max(masked, axis=-1)
        i2 = jnp.argmax(masked, axis=-1).astype(jnp.int32)
        e2 = jnp.exp(m2 - m1)
        denom = 1.0 + e2
        w_first = 1.0 / denom
        w_second = e2 / denom
        iota2 = jax.lax.broadcasted_iota(jnp.int32, (b, topk), 1)
        wts_ref[...] = jnp.where(iota2 == 0, w_first[:, None], w_second[:, None])
        idx_ref[...] = jnp.where(iota2 == 0, i1[:, None], i2[:, None])
        probs = jax.nn.softmax(logits, axis=-1)
        mean_prob = jnp.mean(probs, axis=0)
        onehot = ((iota_e == i1[:, None]).astype(jnp.float32)
                  + (iota_e == i2[:, None]).astype(jnp.float32))
        usage = jnp.sum(onehot, axis=0)
        mean_usage = usage / (b * topk)
        loss = _LBW * e * jnp.sum(mean_prob * mean_usage)
        loss_ref[...] = loss[None, None]


def _tc_merge(acc_head, sums, W1, b1, W2, b2, s):
    nsplit, B, _ = sums.shape
    D = W1.shape[0]
    E = W2.shape[1]
    TOPK = 2
    nk = _D_SC // _KT
    nk_head = (D - _D_SC) // _KT
    kfn = functools.partial(_tc_merge_kernel, nk=nk, s=s, e=E, topk=TOPK,
                            nsplit=nsplit)
    wts, idx, loss = pl.pallas_call(
        kfn,
        grid=(nk,),
        in_specs=[
            pl.BlockSpec((B, D), lambda j: (0, 0)),
            pl.BlockSpec((nsplit, B, _KT), lambda j: (0, 0, j)),
            pl.BlockSpec((_KT, D), lambda j: (j + nk_head, 0)),
            pl.BlockSpec((1, D), lambda j: (0, 0)),
            pl.BlockSpec((D, E), lambda j: (0, 0)),
            pl.BlockSpec((1, E), lambda j: (0, 0)),
        ],
        out_specs=[
            pl.BlockSpec((B, TOPK), lambda j: (0, 0)),
            pl.BlockSpec((B, TOPK), lambda j: (0, 0)),
            pl.BlockSpec((1, 1), lambda j: (0, 0)),
        ],
        out_shape=[
            jax.ShapeDtypeStruct((B, TOPK), jnp.float32),
            jax.ShapeDtypeStruct((B, TOPK), jnp.int32),
            jax.ShapeDtypeStruct((1, 1), jnp.float32),
        ],
        scratch_shapes=[pltpu.VMEM((B, D), jnp.float32)],
        compiler_params=pltpu.CompilerParams(
            dimension_semantics=("arbitrary",),
        ),
    )(acc_head, sums, W1, b1.reshape(1, D), W2, b2.reshape(1, E))
    return wts, idx, loss.reshape(())


def kernel(x, W1, b1, W2, b2):
    B, S, D = x.shape
    sums_tail = _sc_colsums_tail(x)
    acc_head = _tc_head(x, W1, S)
    return _tc_merge(acc_head, sums_tail, W1, b1, W2, b2, S)


# speedup vs baseline: 1.9768x; 1.5050x over previous
import functools

import jax
import jax.numpy as jnp
from jax.experimental import pallas as pl
from jax.experimental.pallas import tpu as pltpu

_LBW = 0.01
_KT = 256


def _gate_kernel(x_ref, w1_ref, b1_ref, w2_ref, b2_ref,
                 wts_ref, idx_ref, loss_ref, acc_ref, *, nk, s, e, topk):
    k = pl.program_id(0)

    @pl.when(k == 0)
    def _init():
        acc_ref[...] = jnp.zeros_like(acc_ref)

    r = jnp.sum(x_ref[...], axis=1) * (1.0 / s)
    acc_ref[...] += jnp.dot(r, w1_ref[...],
                            preferred_element_type=jnp.float32)

    @pl.when(k == nk - 1)
    def _epilogue():
        h = acc_ref[...] + b1_ref[...]
        h = h * jax.nn.sigmoid(h)
        logits = jnp.dot(h, w2_ref[...],
                         preferred_element_type=jnp.float32) + b2_ref[...]
        b = logits.shape[0]
        iota_e = jax.lax.broadcasted_iota(jnp.int32, (b, e), 1)
        m1 = jnp.max(logits, axis=-1)
        i1 = jnp.argmax(logits, axis=-1).astype(jnp.int32)
        masked = jnp.where(iota_e == i1[:, None], -jnp.inf, logits)
        m2 = jnp.max(masked, axis=-1)
        i2 = jnp.argmax(masked, axis=-1).astype(jnp.int32)
        e2 = jnp.exp(m2 - m1)
        denom = 1.0 + e2
        w_first = 1.0 / denom
        w_second = e2 / denom
        iota2 = jax.lax.broadcasted_iota(jnp.int32, (b, topk), 1)
        wts_ref[...] = jnp.where(iota2 == 0, w_first[:, None], w_second[:, None])
        idx_ref[...] = jnp.where(iota2 == 0, i1[:, None], i2[:, None])
        probs = jax.nn.softmax(logits, axis=-1)
        mean_prob = jnp.mean(probs, axis=0)
        onehot = ((iota_e == i1[:, None]).astype(jnp.float32)
                  + (iota_e == i2[:, None]).astype(jnp.float32))
        usage = jnp.sum(onehot, axis=0)
        mean_usage = usage / (b * topk)
        loss = _LBW * e * jnp.sum(mean_prob * mean_usage)
        loss_ref[...] = loss[None, None]


def kernel(x, W1, b1, W2, b2):
    B, S, D = x.shape
    E = W2.shape[1]
    TOPK = 2
    nk = D // _KT

    grid = (nk,)
    kfn = functools.partial(_gate_kernel, nk=nk, s=S, e=E, topk=TOPK)
    wts, idx, loss = pl.pallas_call(
        kfn,
        grid=grid,
        in_specs=[
            pl.BlockSpec((B, S, _KT), lambda k: (0, 0, k)),
            pl.BlockSpec((_KT, D), lambda k: (k, 0)),
            pl.BlockSpec((1, D), lambda k: (0, 0)),
            pl.BlockSpec((D, E), lambda k: (0, 0)),
            pl.BlockSpec((1, E), lambda k: (0, 0)),
        ],
        out_specs=[
            pl.BlockSpec((B, TOPK), lambda k: (0, 0)),
            pl.BlockSpec((B, TOPK), lambda k: (0, 0)),
            pl.BlockSpec((1, 1), lambda k: (0, 0)),
        ],
        out_shape=[
            jax.ShapeDtypeStruct((B, TOPK), jnp.float32),
            jax.ShapeDtypeStruct((B, TOPK), jnp.int32),
            jax.ShapeDtypeStruct((1, 1), jnp.float32),
        ],
        scratch_shapes=[pltpu.VMEM((B, D), jnp.float32)],
        compiler_params=pltpu.CompilerParams(
            dimension_semantics=("arbitrary",),
        ),
    )(x, W1, b1.reshape(1, D), W2, b2.reshape(1, E))
    return wts, idx, loss.reshape(())
